# baseline (speedup 1.0000x reference)
import functools

import jax
import jax.numpy as jnp
from jax import lax
from jax.experimental import pallas as pl
from jax.experimental.pallas import tpu as pltpu
from jax.experimental.pallas import tpu_sc as plsc

NUM_TASKS = 3
NUM_TABLES = 3
BATCH = 16384
NC = 2
NS = 16
NW = NC * NS
CHUNK = BATCH // NW
L = 16
ROW_W = 8
ROW_SHIFT = 3
IPAD = 4
OUT_WORDS = NUM_TASKS * IPAD * BATCH
HASHES = (1000000, 100000, 1000)
BASES = tuple(
    sum(HASHES) * n + sum(HASHES[:i])
    for n in range(NUM_TASKS) for i in range(NUM_TABLES)
)
CAT_PAD = 1000
CAT_ROWS = (sum(HASHES) * NUM_TASKS + CAT_PAD) // ROW_W


def _sc_body(idx0, idx1, idx2, wcat, out_hbm,
             idx_v0, idx_v1, idx_v2,
             r0, r1, r2, r3, r4, r5, r6, r7, r8,
             g0, g1, g2, g3, g4, g5, g6, g7, g8,
             o0, o1, o2, o3, o4, o5, o6, o7, o8,
             s_idx, sw0, sw1, sw2, sw3, sw4, sw5, sw6, sw7, sw8, s_out):
    c = lax.axis_index("c")
    s = lax.axis_index("s")
    wid = s * NC + c
    base = wid * CHUNK

    idx_vmem = (idx_v0, idx_v1, idx_v2)
    row_v = (r0, r1, r2, r3, r4, r5, r6, r7, r8)
    gath = (g0, g1, g2, g3, g4, g5, g6, g7, g8)
    out_v = (o0, o1, o2, o3, o4, o5, o6, o7, o8)
    sem_w = (sw0, sw1, sw2, sw3, sw4, sw5, sw6, sw7, sw8)

    idx_descs = [
        pltpu.async_copy(h.at[pl.ds(base, CHUNK)], idx_vmem[i], s_idx)
        for i, h in enumerate((idx0, idx1, idx2))
    ]
    for d in idx_descs:
        d.wait()

    wide_descs = []
    for j in range(NUM_TASKS * NUM_TABLES):
        i = j % NUM_TABLES

        def setrows(r, carry, i=i, j=j):
            row_v[j][pl.ds(r * L, L)] = (
                (idx_vmem[i][pl.ds(r * L, L)] + BASES[j]) >> ROW_SHIFT)
            return carry
        lax.fori_loop(0, CHUNK // L, setrows, 0)
        wide_descs.append(
            pltpu.async_copy(wcat.at[row_v[j]], gath[j], sem_w[j]))

    iota = lax.iota(jnp.int32, L)
    out_descs = []
    for j in range(NUM_TASKS * NUM_TABLES):
        n, i = divmod(j, NUM_TABLES)
        wide_descs[j].wait()

        def extract(r, carry, i=i, j=j):
            lanes = (idx_vmem[i][pl.ds(r * L, L)] + BASES[j]) & (ROW_W - 1)
            out_v[j][pl.ds(r * L, L)] = plsc.load_gather(
                gath[j], [r * L + iota, lanes])
            return carry
        lax.fori_loop(0, CHUNK // L, extract, 0)
        out_descs.append(pltpu.async_copy(
            out_v[j],
            out_hbm.at[pl.ds(n * IPAD * BATCH + i * BATCH + base, CHUNK)],
            s_out))

    for d in out_descs:
        d.wait()


_sc_call = functools.partial(
    pl.kernel,
    out_type=jax.ShapeDtypeStruct((OUT_WORDS,), jnp.float32),
    mesh=plsc.VectorSubcoreMesh(core_axis_name="c", subcore_axis_name="s",
                                num_cores=NC, num_subcores=NS),
    compiler_params=pltpu.CompilerParams(needs_layout_passes=False,
                                         use_tc_tiling_on_sc=False),
    scratch_types=[
        *[pltpu.VMEM((CHUNK,), jnp.int32) for _ in range(3)],
        *[pltpu.VMEM((CHUNK,), jnp.int32) for _ in range(9)],
        *[pltpu.VMEM((CHUNK, ROW_W), jnp.float32) for _ in range(9)],
        *[pltpu.VMEM((CHUNK,), jnp.float32) for _ in range(9)],
        *[pltpu.SemaphoreType.DMA for _ in range(11)],
    ],
)(_sc_body)


@jax.jit
def kernel(
    indices_0, offsets_0,
    indices_1, offsets_1,
    indices_2, offsets_2,
    W_0_0, W_0_1, W_0_2,
    W_1_0, W_1_1, W_1_2,
    W_2_0, W_2_1, W_2_2,
) -> jnp.ndarray:
    del offsets_0, offsets_1, offsets_2
    wcat = jnp.concatenate(
        [W_0_0, W_0_1, W_0_2, W_1_0, W_1_1, W_1_2, W_2_0, W_2_1, W_2_2,
         jnp.zeros((CAT_PAD, 1), jnp.float32)],
        axis=0).reshape(CAT_ROWS, ROW_W)
    flat = _sc_call(indices_0, indices_1, indices_2, wcat)
    return flat.reshape(NUM_TASKS, IPAD, BATCH)[:, :NUM_TABLES, :].transpose(
        0, 2, 1)


# speedup vs baseline: 2.5174x; 2.5174x over previous
import functools

import jax
import jax.numpy as jnp
from jax import lax
from jax.experimental import pallas as pl
from jax.experimental.pallas import tpu as pltpu
from jax.experimental.pallas import tpu_sc as plsc

NUM_TASKS = 3
NUM_TABLES = 3
BATCH = 16384
NC = 2
NS = 16
NW = NC * NS
CHUNK = BATCH // NW
L = 16
ROW_W = 8
ROW_SHIFT = 3
SMALL_H = 1000
IPAD = 4
OUT_WORDS = NUM_TASKS * IPAD * BATCH


def _sc_body(idx0, idx1, idx2,
             wa0, wa1, wb0, wb1, wc0, wc1,
             ws0, ws1, ws2,
             out_hbm,
             idx_v0, idx_v1, idx_v2, row_v0, row_v1,
             g0, g1, g2, g3, g4, g5,
             sv0, sv1, sv2,
             o0, o1, o2, o3, o4, o5, o6, o7, o8,
             s_idx, s_sm, sw0, sw1, sw2, sw3, sw4, sw5, s_out):
    c = lax.axis_index("c")
    s = lax.axis_index("s")
    wid = s * NC + c
    base = wid * CHUNK

    idx_vmem = (idx_v0, idx_v1, idx_v2)
    wide = ((wa0, wa1), (wb0, wb1), (wc0, wc1))
    small = (ws0, ws1, ws2)
    small_v = (sv0, sv1, sv2)
    row_v = (row_v0, row_v1)
    gath = ((g0, g1), (g2, g3), (g4, g5))
    sem_w = ((sw0, sw1), (sw2, sw3), (sw4, sw5))
    out_v = ((o0, o1), (o3, o4), (o6, o7))
    out_sv = (o2, o5, o8)

    idx_descs = [
        pltpu.async_copy(h.at[pl.ds(base, CHUNK)], idx_vmem[i], s_idx)
        for i, h in enumerate((idx0, idx1, idx2))
    ]
    sm_descs = [
        pltpu.async_copy(small[n], small_v[n], s_sm)
        for n in range(NUM_TASKS)
    ]
    for d in idx_descs:
        d.wait()

    for i in range(2):
        def setrows(r, carry, i=i):
            row_v[i][pl.ds(r * L, L)] = (
                idx_vmem[i][pl.ds(r * L, L)] >> ROW_SHIFT)
            return carry
        lax.fori_loop(0, CHUNK // L, setrows, 0)

    wide_descs = [
        [pltpu.async_copy(wide[n][i].at[row_v[i]], gath[n][i], sem_w[n][i])
         for i in range(2)]
        for n in range(NUM_TASKS)
    ]

    iota = lax.iota(jnp.int32, L)
    out_descs = []

    for d in sm_descs:
        d.wait()
    for n in range(NUM_TASKS):
        def extract_small(r, carry, n=n):
            rows = idx_vmem[2][pl.ds(r * L, L)]
            out_sv[n][pl.ds(r * L, L)] = plsc.load_gather(
                small_v[n], [rows, rows * 0])
            return carry
        lax.fori_loop(0, CHUNK // L, extract_small, 0)
        out_descs.append(pltpu.async_copy(
            out_sv[n],
            out_hbm.at[pl.ds(n * IPAD * BATCH + 2 * BATCH + base, CHUNK)],
            s_out))

    for n in range(NUM_TASKS):
        for i in range(2):
            wide_descs[n][i].wait()

            def extract(r, carry, n=n, i=i):
                lanes = idx_vmem[i][pl.ds(r * L, L)] & (ROW_W - 1)
                out_v[n][i][pl.ds(r * L, L)] = plsc.load_gather(
                    gath[n][i], [r * L + iota, lanes])
                return carry
            lax.fori_loop(0, CHUNK // L, extract, 0)
            out_descs.append(pltpu.async_copy(
                out_v[n][i],
                out_hbm.at[pl.ds(n * IPAD * BATCH + i * BATCH + base, CHUNK)],
                s_out))

    for d in out_descs:
        d.wait()


_sc_call = functools.partial(
    pl.kernel,
    out_type=jax.ShapeDtypeStruct((OUT_WORDS,), jnp.float32),
    mesh=plsc.VectorSubcoreMesh(core_axis_name="c", subcore_axis_name="s",
                                num_cores=NC, num_subcores=NS),
    compiler_params=pltpu.CompilerParams(needs_layout_passes=False,
                                         use_tc_tiling_on_sc=False),
    scratch_types=[
        *[pltpu.VMEM((CHUNK,), jnp.int32) for _ in range(5)],
        *[pltpu.VMEM((CHUNK, ROW_W), jnp.float32) for _ in range(6)],
        *[pltpu.VMEM((SMALL_H, 1), jnp.float32) for _ in range(3)],
        *[pltpu.VMEM((CHUNK,), jnp.float32) for _ in range(9)],
        *[pltpu.SemaphoreType.DMA for _ in range(9)],
    ],
)(_sc_body)


def _wide(w):
    return w.T.reshape(-1, ROW_W)


@jax.jit
def kernel(
    indices_0, offsets_0,
    indices_1, offsets_1,
    indices_2, offsets_2,
    W_0_0, W_0_1, W_0_2,
    W_1_0, W_1_1, W_1_2,
    W_2_0, W_2_1, W_2_2,
) -> jnp.ndarray:
    del offsets_0, offsets_1, offsets_2
    flat = _sc_call(
        indices_0, indices_1, indices_2,
        _wide(W_0_0), _wide(W_0_1),
        _wide(W_1_0), _wide(W_1_1),
        _wide(W_2_0), _wide(W_2_1),
        W_0_2, W_1_2, W_2_2,
    )
    return flat.reshape(NUM_TASKS, IPAD, BATCH)[:, :NUM_TABLES, :].transpose(
        0, 2, 1)
